# baseline (speedup 1.0000x reference)
import functools

import jax
import jax.numpy as jnp
from jax import lax
from jax.experimental import pallas as pl
from jax.experimental.pallas import tpu as pltpu
from jax.experimental.pallas import tpu_sc as plsc

N = 320000
D = 128
NC = 2
NS = 16
NW = NC * NS

R_SC = 256000
R_TC = N - R_SC

ROWS_PER_W = R_SC // NW
C = 200
NBUF = 5
NCHUNK = ROWS_PER_W // C
NGROUP = NCHUNK // NBUF
LANES = 16
JL = D // LANES


def _sc_partial_sums(x_hbm, out_hbm, bufs, acc_v, sems):
    wid = lax.axis_index("s") * NC + lax.axis_index("c")
    base = wid * ROWS_PER_W

    for b in range(NBUF):
        pltpu.async_copy(x_hbm.at[pl.ds(base + b * C, C)], bufs[b], sems[b])

    def group_body(g, accs):
        for b in range(NBUF):
            chunk = g * NBUF + b
            pltpu.make_async_copy(
                x_hbm.at[pl.ds(base, C)], bufs[b], sems[b]
            ).wait()

            def row_body(r, a):
                return tuple(
                    a[j] + bufs[b][r, pl.ds(j * LANES, LANES)]
                    for j in range(JL)
                )

            accs = lax.fori_loop(0, C, row_body, accs, unroll=2)

            nxt = chunk + NBUF

            @pl.when(nxt < NCHUNK)
            def _():
                pltpu.async_copy(
                    x_hbm.at[pl.ds(base + nxt * C, C)], bufs[b], sems[b]
                )

        return accs

    zeros = tuple(jnp.zeros((LANES,), jnp.float32) for _ in range(JL))
    accs = lax.fori_loop(0, NGROUP, group_body, zeros)

    for j in range(JL):
        acc_v[0, pl.ds(j * LANES, LANES)] = accs[j]
    pltpu.sync_copy(acc_v, out_hbm.at[wid])


@functools.partial(
    pl.kernel,
    out_type=jax.ShapeDtypeStruct((NW, 1, D), jnp.float32),
    mesh=plsc.VectorSubcoreMesh(core_axis_name="c", subcore_axis_name="s"),
    scratch_types=(
        [pltpu.VMEM((C, D), jnp.float32) for _ in range(NBUF)]
        + [pltpu.VMEM((1, D), jnp.float32)]
        + [pltpu.SemaphoreType.DMA for _ in range(NBUF)]
    ),
)
def _sc_sum_kernel(x_hbm, out_hbm, *refs):
    bufs = refs[:NBUF]
    acc_v = refs[NBUF]
    sems = refs[NBUF + 1:]
    _sc_partial_sums(x_hbm, out_hbm, bufs, acc_v, sems)


BR = 1000
TC_STEPS = R_TC // BR
TC_OFF = R_SC // BR


def _tc_body(x_ref, o_ref):
    @pl.when(pl.program_id(0) == 0)
    def _():
        o_ref[...] = jnp.zeros_like(o_ref)

    o_ref[...] += jnp.sum(x_ref[...], axis=0, keepdims=True)


_tc_tail_sum = pl.pallas_call(
    _tc_body,
    grid=(TC_STEPS,),
    in_specs=[pl.BlockSpec((BR, D), lambda i: (TC_OFF + i, 0))],
    out_specs=pl.BlockSpec((1, D), lambda i: (0, 0)),
    out_shape=jax.ShapeDtypeStruct((1, D), jnp.float32),
)


def _combine_body(p_ref, t_ref, o_ref):
    o_ref[...] = jnp.sum(p_ref[...], axis=0) + t_ref[...]


_combine = pl.pallas_call(
    _combine_body,
    out_shape=jax.ShapeDtypeStruct((1, D), jnp.float32),
)


_tc_full_sum = pl.pallas_call(
    _tc_body,
    grid=(N // 2000,),
    in_specs=[pl.BlockSpec((2000, D), lambda i: (i, 0))],
    out_specs=pl.BlockSpec((1, D), lambda i: (0, 0)),
    out_shape=jax.ShapeDtypeStruct((1, D), jnp.float32),
)


def kernel(x):
    return _tc_full_sum(x)


# speedup vs baseline: 2.5448x; 2.5448x over previous
import functools

import jax
import jax.numpy as jnp
from jax import lax
from jax.experimental import pallas as pl
from jax.experimental.pallas import tpu as pltpu
from jax.experimental.pallas import tpu_sc as plsc

N = 320000
D = 128
NC = 2
NS = 16
NW = NC * NS

R_SC = 256000
R_TC = N - R_SC

ROWS_PER_W = R_SC // NW
C = 200
NBUF = 5
NCHUNK = ROWS_PER_W // C
NGROUP = NCHUNK // NBUF
LANES = 16
JL = D // LANES


def _sc_partial_sums(x_hbm, out_hbm, bufs, acc_v, sems):
    wid = lax.axis_index("s") * NC + lax.axis_index("c")
    base = wid * ROWS_PER_W

    for b in range(NBUF):
        pltpu.async_copy(x_hbm.at[pl.ds(base + b * C, C)], bufs[b], sems[b])

    def group_body(g, accs):
        for b in range(NBUF):
            chunk = g * NBUF + b
            pltpu.make_async_copy(
                x_hbm.at[pl.ds(base, C)], bufs[b], sems[b]
            ).wait()

            def row_body(r, a):
                return tuple(
                    a[j] + bufs[b][r, pl.ds(j * LANES, LANES)]
                    for j in range(JL)
                )

            accs = lax.fori_loop(0, C, row_body, accs, unroll=2)

            nxt = chunk + NBUF

            @pl.when(nxt < NCHUNK)
            def _():
                pltpu.async_copy(
                    x_hbm.at[pl.ds(base + nxt * C, C)], bufs[b], sems[b]
                )

        return accs

    zeros = tuple(jnp.zeros((LANES,), jnp.float32) for _ in range(JL))
    accs = lax.fori_loop(0, NGROUP, group_body, zeros)

    for j in range(JL):
        acc_v[0, pl.ds(j * LANES, LANES)] = accs[j]
    pltpu.sync_copy(acc_v, out_hbm.at[wid])


@functools.partial(
    pl.kernel,
    out_type=jax.ShapeDtypeStruct((NW, 1, D), jnp.float32),
    mesh=plsc.VectorSubcoreMesh(core_axis_name="c", subcore_axis_name="s"),
    scratch_types=(
        [pltpu.VMEM((C, D), jnp.float32) for _ in range(NBUF)]
        + [pltpu.VMEM((1, D), jnp.float32)]
        + [pltpu.SemaphoreType.DMA for _ in range(NBUF)]
    ),
)
def _sc_sum_kernel(x_hbm, out_hbm, *refs):
    bufs = refs[:NBUF]
    acc_v = refs[NBUF]
    sems = refs[NBUF + 1:]
    _sc_partial_sums(x_hbm, out_hbm, bufs, acc_v, sems)


BR = 1000
TC_STEPS = R_TC // BR
TC_OFF = R_SC // BR


def _tc_body(x_ref, o_ref):
    @pl.when(pl.program_id(0) == 0)
    def _():
        o_ref[...] = jnp.zeros_like(o_ref)

    o_ref[...] += jnp.sum(x_ref[...], axis=0, keepdims=True)


_tc_tail_sum = pl.pallas_call(
    _tc_body,
    grid=(TC_STEPS,),
    in_specs=[pl.BlockSpec((BR, D), lambda i: (TC_OFF + i, 0))],
    out_specs=pl.BlockSpec((1, D), lambda i: (0, 0)),
    out_shape=jax.ShapeDtypeStruct((1, D), jnp.float32),
)


def _combine_body(p_ref, t_ref, o_ref):
    o_ref[...] = jnp.sum(p_ref[...], axis=0) + t_ref[...]


_combine = pl.pallas_call(
    _combine_body,
    out_shape=jax.ShapeDtypeStruct((1, D), jnp.float32),
)


BRF = 32000


def _tc_full_body(x_ref, o_ref, acc_ref):
    @pl.when(pl.program_id(0) == 0)
    def _():
        acc_ref[...] = jnp.zeros_like(acc_ref)

    xb = x_ref[...].reshape(BRF // 64, 64, D)
    acc_ref[...] += jnp.sum(xb, axis=0)

    @pl.when(pl.program_id(0) == N // BRF - 1)
    def _():
        o_ref[...] = jnp.sum(acc_ref[...], axis=0, keepdims=True)


_tc_full_sum = pl.pallas_call(
    _tc_full_body,
    grid=(N // BRF,),
    in_specs=[pl.BlockSpec((BRF, D), lambda i: (i, 0))],
    out_specs=pl.BlockSpec((1, D), lambda i: (0, 0)),
    out_shape=jax.ShapeDtypeStruct((1, D), jnp.float32),
    scratch_shapes=[pltpu.VMEM((64, D), jnp.float32)],
)


def kernel(x):
    return _tc_full_sum(x)
